# baseline (speedup 1.0000x reference)
import math

import jax
import jax.numpy as jnp
import numpy as np
from jax.experimental import pallas as pl
from jax.experimental.pallas import tpu as pltpu


S = 2048
DIM = 768
NH = 12
HD = DIM // NH
HP = 128
DIMP = NH * HP
NE = 8
HALF = HD // 2
RBLK = 256
ABLK = 4
LN_EPS = 1e-5
QSCALE = math.log2(math.e) / math.sqrt(HD)
INV_DIM = 1.0 / DIM


def _swap_halves(v):
    lane = jax.lax.broadcasted_iota(jnp.int32, v.shape, 1)
    left = pltpu.roll(v, v.shape[1] - HALF, axis=1)
    right = pltpu.roll(v, HALF, axis=1)
    return jnp.where(jnp.bitwise_and(lane, HD - 1) < HALF, left, right)


def _moe_attn_kernel(x_ref, cs_ref, gw_ref, gb_ref,
                     wq_ref, wk_ref, wv_ref, wo_ref,
                     qg_ref, qb_ref, kg_ref, kb_ref,
                     out_ref, qr, kr, vb, acc, cosb, sinb):
    e = pl.program_id(0)

    def ln(p, g_ref, b_ref):
        mu = jnp.sum(p, axis=-1, keepdims=True) * INV_DIM
        ex2 = jnp.sum(p * p, axis=-1, keepdims=True) * INV_DIM
        var = ex2 - mu * mu
        return (p - mu) * jax.lax.rsqrt(var + LN_EPS) * g_ref[0] + b_ref[0]

    @pl.when(e == 0)
    def _():
        lane = jax.lax.broadcasted_iota(jnp.int32, (S, DIMP), 1)
        vb[...] = jnp.where(jnp.bitwise_and(lane, HP - 1) == HD,
                            1.0, 0.0).astype(jnp.bfloat16)
        cs = cs_ref[...]
        sw = _swap_halves(cs)
        lane = jax.lax.broadcasted_iota(jnp.int32, cs.shape, 1)
        first = jnp.bitwise_and(lane, HD - 1) < HALF
        cosb[...] = jnp.where(first, cs, sw)
        sinb[...] = jnp.where(first, -sw, cs)

    def qkv_body(r, carry):
        rows = pl.ds(r * RBLK, RBLK)
        xc = x_ref[rows, :]
        cos = cosb[rows, :].astype(jnp.float32)
        sin = sinb[rows, :].astype(jnp.float32)

        q = ln(jnp.dot(xc, wq_ref[0], preferred_element_type=jnp.float32),
               qg_ref, qb_ref)
        qr[rows, :] = ((q * cos + _swap_halves(q) * sin)
                       * QSCALE).astype(jnp.bfloat16)
        k = ln(jnp.dot(xc, wk_ref[0], preferred_element_type=jnp.float32),
               kg_ref, kb_ref)
        kr[rows, :] = (k * cos + _swap_halves(k) * sin).astype(jnp.bfloat16)
        v = jnp.dot(xc, wv_ref[0],
                    preferred_element_type=jnp.float32).astype(jnp.bfloat16)
        for h in range(NH):
            vb[rows, h * HP:h * HP + HD] = v[:, h * HD:(h + 1) * HD]
        return carry

    jax.lax.fori_loop(0, S // RBLK, qkv_body, 0)

    for h in range(NH):
        kh = kr[:, h * HD:(h + 1) * HD]
        vh = vb[:, h * HP:(h + 1) * HP]

        def attn_body(i, carry, kh=kh, vh=vh, h=h, nb=ABLK):
            hs = slice(h * HD, (h + 1) * HD)
            dims = (((1,), (1,)), ((), ()))
            rs = [pl.ds((i * nb + j) * RBLK, RBLK) for j in range(nb)]
            ss = [jax.lax.dot_general(qr[r, hs], kh, dims,
                                      preferred_element_type=jnp.float32)
                  for r in rs]
            ms = [jnp.max(s, axis=-1, keepdims=True) for s in ss]
            ps = [jnp.exp2(s - m).astype(jnp.bfloat16)
                  for s, m in zip(ss, ms)]
            os = [jnp.dot(p, vh, preferred_element_type=jnp.float32)
                  for p in ps]
            for r, o in zip(rs, os):
                acc[r, hs] = (o[:, :HD] / o[:, HD:HD + 1]).astype(jnp.bfloat16)
            return carry

        jax.lax.fori_loop(0, S // (ABLK * RBLK), attn_body, 0)

    @pl.when(e == 0)
    def _():
        out_ref[...] = jnp.zeros_like(out_ref)

    nel = gw_ref.shape[1]

    def out_body(r, carry):
        rows = pl.ds(r * RBLK, RBLK)
        gall = jax.nn.sigmoid(
            jnp.dot(x_ref[rows, :], gw_ref[...],
                    preferred_element_type=jnp.float32) + gb_ref[...])
        eoh = jax.lax.broadcasted_iota(jnp.int32, (1, nel), 1) == e
        gcol = jnp.sum(jnp.where(eoh, gall, 0.0), axis=1, keepdims=True)
        o = jnp.dot(acc[rows, :], wo_ref[0],
                    preferred_element_type=jnp.float32)
        out_ref[rows, :] += o * gcol
        return carry

    jax.lax.fori_loop(0, S // RBLK, out_body, 0)


def _build_perm():
    perm = np.zeros(DIM, dtype=np.int32)
    for h in range(NH):
        base = h * HD
        for j in range(HALF):
            perm[base + j] = base + 2 * j
            perm[base + HALF + j] = base + 2 * j + 1
    return perm


_PERM = _build_perm()


def _deint_w(w):
    return w[:, :, _PERM]


def _deint_v(v):
    return v[:, _PERM].reshape(v.shape[0], 1, DIM)


def kernel(x, freqs_cis, Wq, Wk, Wv, Wo, q_g, q_b, k_g, k_b, gate_w, gate_b):
    out = _run_core(x, freqs_cis, Wq, Wk, Wv, Wo, q_g, q_b, k_g, k_b,
                    gate_w, gate_b.reshape(1, NE))
    return out[None]


def _run_core(x, freqs_cis, Wq, Wk, Wv, Wo, q_g, q_b, k_g, k_b,
              gate_w, gate_b):
    nel = Wq.shape[0]

    xb = x[0].astype(jnp.bfloat16)
    wq = _deint_w(Wq).astype(jnp.bfloat16)
    wk = _deint_w(Wk).astype(jnp.bfloat16)
    wv = Wv.astype(jnp.bfloat16)
    wo = Wo.astype(jnp.bfloat16)
    qg = _deint_v(q_g)
    qb = _deint_v(q_b)
    kg = _deint_v(k_g)
    kb = _deint_v(k_b)
    gw = gate_w.astype(jnp.bfloat16)
    gb = gate_b

    cos_ = freqs_cis[:, :, 0]
    sin_ = freqs_cis[:, :, 1]
    csf = jnp.tile(jnp.concatenate([cos_, sin_], axis=1),
                   (1, NH)).astype(jnp.bfloat16)
    full = lambda *_: (0, 0)
    per_e = lambda e: (e, 0, 0)

    return pl.pallas_call(
        _moe_attn_kernel,
        grid=(nel,),
        in_specs=[
            pl.BlockSpec((S, DIM), full),
            pl.BlockSpec((S, DIM), full),
            pl.BlockSpec((DIM, nel), full),
            pl.BlockSpec((1, nel), full),
            pl.BlockSpec((1, DIM, DIM), per_e),
            pl.BlockSpec((1, DIM, DIM), per_e),
            pl.BlockSpec((1, DIM, DIM), per_e),
            pl.BlockSpec((1, DIM, DIM), per_e),
            pl.BlockSpec((1, 1, DIM), per_e),
            pl.BlockSpec((1, 1, DIM), per_e),
            pl.BlockSpec((1, 1, DIM), per_e),
            pl.BlockSpec((1, 1, DIM), per_e),
        ],
        out_specs=pl.BlockSpec((S, DIM), full),
        out_shape=jax.ShapeDtypeStruct((S, DIM), jnp.float32),
        scratch_shapes=[
            pltpu.VMEM((S, DIM), jnp.bfloat16),
            pltpu.VMEM((S, DIM), jnp.bfloat16),
            pltpu.VMEM((S, DIMP), jnp.bfloat16),
            pltpu.VMEM((S, DIM), jnp.bfloat16),
            pltpu.VMEM((S, DIM), jnp.bfloat16),
            pltpu.VMEM((S, DIM), jnp.bfloat16),
        ],
        compiler_params=pltpu.CompilerParams(
            dimension_semantics=("arbitrary",)),
    )(xb, csf, gw, gb, wq, wk, wv, wo, qg, qb, kg, kb)


# speedup vs baseline: 1.0152x; 1.0152x over previous
import math

import jax
import jax.numpy as jnp
import numpy as np
from jax.experimental import pallas as pl
from jax.experimental.pallas import tpu as pltpu


S = 2048
DIM = 768
NH = 12
HD = DIM // NH
HP = 128
DIMP = NH * HP
NE = 8
HALF = HD // 2
RBLK = 256
ABLK = 2
LN_EPS = 1e-5
QSCALE = math.log2(math.e) / math.sqrt(HD)
INV_DIM = 1.0 / DIM


def _swap_halves(v):
    lane = jax.lax.broadcasted_iota(jnp.int32, v.shape, 1)
    left = pltpu.roll(v, v.shape[1] - HALF, axis=1)
    right = pltpu.roll(v, HALF, axis=1)
    return jnp.where(jnp.bitwise_and(lane, HD - 1) < HALF, left, right)


def _moe_attn_kernel(x_ref, cs_ref, gw_ref, gb_ref,
                     wq_ref, wk_ref, wv_ref, wo_ref,
                     qg_ref, qb_ref, kg_ref, kb_ref,
                     out_ref, qr, kr, vb, acc, cosb, sinb):
    e = pl.program_id(0)

    def ln(p, g_ref, b_ref):
        mu = jnp.sum(p, axis=-1, keepdims=True) * INV_DIM
        ex2 = jnp.sum(p * p, axis=-1, keepdims=True) * INV_DIM
        var = ex2 - mu * mu
        return (p - mu) * jax.lax.rsqrt(var + LN_EPS) * g_ref[0] + b_ref[0]

    @pl.when(e == 0)
    def _():
        lane = jax.lax.broadcasted_iota(jnp.int32, (S, DIMP), 1)
        vb[...] = jnp.where(jnp.bitwise_and(lane, HP - 1) == HD,
                            1.0, 0.0).astype(jnp.bfloat16)
        cs = cs_ref[...]
        sw = _swap_halves(cs)
        lane = jax.lax.broadcasted_iota(jnp.int32, cs.shape, 1)
        first = jnp.bitwise_and(lane, HD - 1) < HALF
        cosb[...] = jnp.where(first, cs, sw)
        sinb[...] = jnp.where(first, -sw, cs)

    def qkv_body(r, carry):
        rows = pl.ds(r * RBLK, RBLK)
        xc = x_ref[rows, :]
        cos = cosb[rows, :].astype(jnp.float32)
        sin = sinb[rows, :].astype(jnp.float32)

        pq = jnp.dot(xc, wq_ref[0], preferred_element_type=jnp.float32)
        pk = jnp.dot(xc, wk_ref[0], preferred_element_type=jnp.float32)
        pv = jnp.dot(xc, wv_ref[0], preferred_element_type=jnp.float32)
        q = ln(pq, qg_ref, qb_ref)
        k = ln(pk, kg_ref, kb_ref)
        qr[rows, :] = ((q * cos + _swap_halves(q) * sin)
                       * QSCALE).astype(jnp.bfloat16)
        kr[rows, :] = (k * cos + _swap_halves(k) * sin).astype(jnp.bfloat16)
        v = pv.astype(jnp.bfloat16)
        for h in range(NH):
            vb[rows, h * HP:h * HP + HD] = v[:, h * HD:(h + 1) * HD]
        return carry

    jax.lax.fori_loop(0, S // RBLK, qkv_body, 0)

    for h in range(NH):
        kh = kr[:, h * HD:(h + 1) * HD]
        vh = vb[:, h * HP:(h + 1) * HP]

        def attn_body(i, carry, kh=kh, vh=vh, h=h, nb=ABLK):
            hs = slice(h * HD, (h + 1) * HD)
            dims = (((1,), (1,)), ((), ()))
            rs = [pl.ds((i * nb + j) * RBLK, RBLK) for j in range(nb)]
            ss = [jax.lax.dot_general(qr[r, hs], kh, dims,
                                      preferred_element_type=jnp.float32)
                  for r in rs]
            ms = [jnp.max(s, axis=-1, keepdims=True) for s in ss]
            ps = [jnp.exp2(s - m).astype(jnp.bfloat16)
                  for s, m in zip(ss, ms)]
            os = [jnp.dot(p, vh, preferred_element_type=jnp.float32)
                  for p in ps]
            for r, o in zip(rs, os):
                acc[r, hs] = (o[:, :HD] / o[:, HD:HD + 1]).astype(jnp.bfloat16)
            return carry

        jax.lax.fori_loop(0, S // (ABLK * RBLK), attn_body, 0)

    @pl.when(e == 0)
    def _():
        out_ref[...] = jnp.zeros_like(out_ref)

    nel = gw_ref.shape[1]

    def out_body(r, carry):
        rows = pl.ds(r * RBLK, RBLK)
        gall = jax.nn.sigmoid(
            jnp.dot(x_ref[rows, :], gw_ref[...],
                    preferred_element_type=jnp.float32) + gb_ref[...])
        eoh = jax.lax.broadcasted_iota(jnp.int32, (1, nel), 1) == e
        gcol = jnp.sum(jnp.where(eoh, gall, 0.0), axis=1, keepdims=True)
        o = jnp.dot(acc[rows, :], wo_ref[0],
                    preferred_element_type=jnp.float32)
        out_ref[rows, :] += o * gcol
        return carry

    jax.lax.fori_loop(0, S // RBLK, out_body, 0)


def _build_perm():
    perm = np.zeros(DIM, dtype=np.int32)
    for h in range(NH):
        base = h * HD
        for j in range(HALF):
            perm[base + j] = base + 2 * j
            perm[base + HALF + j] = base + 2 * j + 1
    return perm


_PERM = _build_perm()


def _deint_w(w):
    return w[:, :, _PERM]


def _deint_v(v):
    return v[:, _PERM].reshape(v.shape[0], 1, DIM)


def kernel(x, freqs_cis, Wq, Wk, Wv, Wo, q_g, q_b, k_g, k_b, gate_w, gate_b):
    out = _run_core(x, freqs_cis, Wq, Wk, Wv, Wo, q_g, q_b, k_g, k_b,
                    gate_w, gate_b.reshape(1, NE))
    return out[None]


def _run_core(x, freqs_cis, Wq, Wk, Wv, Wo, q_g, q_b, k_g, k_b,
              gate_w, gate_b):
    nel = Wq.shape[0]

    xb = x[0].astype(jnp.bfloat16)
    wq = _deint_w(Wq).astype(jnp.bfloat16)
    wk = _deint_w(Wk).astype(jnp.bfloat16)
    wv = Wv.astype(jnp.bfloat16)
    wo = Wo.astype(jnp.bfloat16)
    qg = _deint_v(q_g)
    qb = _deint_v(q_b)
    kg = _deint_v(k_g)
    kb = _deint_v(k_b)
    gw = gate_w.astype(jnp.bfloat16)
    gb = gate_b

    cos_ = freqs_cis[:, :, 0]
    sin_ = freqs_cis[:, :, 1]
    csf = jnp.tile(jnp.concatenate([cos_, sin_], axis=1),
                   (1, NH)).astype(jnp.bfloat16)
    full = lambda *_: (0, 0)
    per_e = lambda e: (e, 0, 0)

    return pl.pallas_call(
        _moe_attn_kernel,
        grid=(nel,),
        in_specs=[
            pl.BlockSpec((S, DIM), full),
            pl.BlockSpec((S, DIM), full),
            pl.BlockSpec((DIM, nel), full),
            pl.BlockSpec((1, nel), full),
            pl.BlockSpec((1, DIM, DIM), per_e),
            pl.BlockSpec((1, DIM, DIM), per_e),
            pl.BlockSpec((1, DIM, DIM), per_e),
            pl.BlockSpec((1, DIM, DIM), per_e),
            pl.BlockSpec((1, 1, DIM), per_e),
            pl.BlockSpec((1, 1, DIM), per_e),
            pl.BlockSpec((1, 1, DIM), per_e),
            pl.BlockSpec((1, 1, DIM), per_e),
        ],
        out_specs=pl.BlockSpec((S, DIM), full),
        out_shape=jax.ShapeDtypeStruct((S, DIM), jnp.float32),
        scratch_shapes=[
            pltpu.VMEM((S, DIM), jnp.bfloat16),
            pltpu.VMEM((S, DIM), jnp.bfloat16),
            pltpu.VMEM((S, DIMP), jnp.bfloat16),
            pltpu.VMEM((S, DIM), jnp.bfloat16),
            pltpu.VMEM((S, DIM), jnp.bfloat16),
            pltpu.VMEM((S, DIM), jnp.bfloat16),
        ],
        compiler_params=pltpu.CompilerParams(
            dimension_semantics=("arbitrary",)),
    )(xb, csf, gw, gb, wq, wk, wv, wo, qg, qb, kg, kb)
